# Initial kernel scaffold; baseline (speedup 1.0000x reference)
import jax
import jax.numpy as jnp
from jax.experimental import pallas as pl


def kernel(x, edge_index, W1, b1, W2, b2, W3, b3):
    raise NotImplementedError("write your pallas kernel here")


# speedup vs baseline: 15.7197x; 15.7197x over previous
import functools

import jax
import jax.numpy as jnp
from jax import lax
from jax.experimental import pallas as pl
from jax.experimental.pallas import tpu as pltpu
from jax.experimental.pallas import tpu_sc as plsc

N = 10000
E = 320000
IN_DIM = 128
HID = 128
OUT_DIM = 64
NUM_CLASSES = 16

NPAD = 10240
ROWS_PER_TILE = NPAD // 16
CHUNK = 128
NCHUNKS = 79
EDGES_PER_TILE = NCHUNKS * CHUNK
EPAD = 32 * EDGES_PER_TILE
DUMMY_DST = N

@functools.cache
def _get_mesh():
  return plsc.VectorSubcoreMesh(core_axis_name="c", subcore_axis_name="s")


_SC_PARAMS = pltpu.CompilerParams(use_tc_tiling_on_sc=False)


def _zero_acc_rows(zbuf, acc, base_rows, d):
  def zrow(i, _):
    def zcol(k, _):
      zbuf[i, pl.ds(k * 16, 16)] = jnp.zeros((16,), jnp.float32)
      return 0
    return lax.fori_loop(0, d // 16, zcol, 0)
  lax.fori_loop(0, 64, zrow, 0)

  def zcp(i, _):
    pltpu.sync_copy(zbuf, acc.at[pl.ds(base_rows + i * 64, 64)])
    return 0
  lax.fori_loop(0, ROWS_PER_TILE // 64, zcp, 0)


@functools.cache
def _make_scatter(d):
  @functools.partial(
      pl.kernel,
      mesh=_get_mesh(),
      compiler_params=_SC_PARAMS,
      out_type=jax.ShapeDtypeStruct((2, NPAD, d), jnp.float32),
      scratch_types=[
          pltpu.VMEM((NCHUNKS, CHUNK), jnp.int32),
          pltpu.VMEM((NCHUNKS, CHUNK), jnp.int32),
          pltpu.VMEM((CHUNK, d), jnp.float32),
          pltpu.VMEM((64, d), jnp.float32),
          pltpu.VMEM_SHARED((NPAD, d), jnp.float32),
          pltpu.SemaphoreType.DMA,
      ],
  )
  def scatter_kernel(g_hbm, src_hbm, dst_hbm, out_hbm,
                     src_v, dst_v, rows_v, zbuf, acc, sem):
    c = lax.axis_index("c")
    s = lax.axis_index("s")
    wid = c * 16 + s
    base_rows = s * ROWS_PER_TILE

    _zero_acc_rows(zbuf, acc, base_rows, d)
    pltpu.sync_copy(src_hbm.at[wid], src_v)
    pltpu.sync_copy(dst_hbm.at[wid], dst_v)
    plsc.subcore_barrier()

    def body(j, _):
      pltpu.async_copy(g_hbm.at[src_v.at[j]], rows_v, sem).wait()
      pltpu.sync_copy(rows_v, acc.at[dst_v.at[j]], add=True)
      return 0
    lax.fori_loop(0, NCHUNKS, body, 0)

    plsc.subcore_barrier()

    def ocp(i, _):
      r = base_rows + i * 64
      pltpu.sync_copy(acc.at[pl.ds(r, 64)], out_hbm.at[c, pl.ds(r, 64)])
      return 0
    lax.fori_loop(0, ROWS_PER_TILE // 64, ocp, 0)

  return scatter_kernel


DEGW = 16


@functools.cache
def _make_degree_kernel():
  @functools.partial(
      pl.kernel,
      mesh=_get_mesh(),
      compiler_params=_SC_PARAMS,
      out_type=jax.ShapeDtypeStruct((2, NPAD, DEGW), jnp.float32),
      scratch_types=[
          pltpu.VMEM((NCHUNKS, CHUNK), jnp.int32),
          pltpu.VMEM((CHUNK, DEGW), jnp.float32),
          pltpu.VMEM((64, DEGW), jnp.float32),
          pltpu.VMEM_SHARED((NPAD, DEGW), jnp.float32),
      ],
  )
  def _degree_kernel(dst_hbm, out_hbm, dst_v, ones_v, zbuf, acc):
    c = lax.axis_index("c")
    s = lax.axis_index("s")
    wid = c * 16 + s
    base_rows = s * ROWS_PER_TILE

    _zero_acc_rows(zbuf, acc, base_rows, DEGW)

    def orow(i, _):
      ones_v[i, pl.ds(0, 16)] = jnp.ones((16,), jnp.float32)
      return 0
    lax.fori_loop(0, CHUNK, orow, 0)

    pltpu.sync_copy(dst_hbm.at[wid], dst_v)
    plsc.subcore_barrier()

    def body(j, _):
      pltpu.sync_copy(ones_v, acc.at[dst_v.at[j]], add=True)
      return 0
    lax.fori_loop(0, NCHUNKS, body, 0)

    plsc.subcore_barrier()

    def ocp(i, _):
      r = base_rows + i * 64
      pltpu.sync_copy(acc.at[pl.ds(r, 64)], out_hbm.at[c, pl.ds(r, 64)])
      return 0
    lax.fori_loop(0, ROWS_PER_TILE // 64, ocp, 0)

  return _degree_kernel


def _mm_body(x_ref, w_ref, o_ref):
  o_ref[...] = jnp.dot(x_ref[...], w_ref[...],
                       preferred_element_type=jnp.float32)


def _tc_matmul(x, w):
  return pl.pallas_call(
      _mm_body,
      out_shape=jax.ShapeDtypeStruct((x.shape[0], w.shape[1]), jnp.float32),
  )(x, w)


def _dinv_scale_body(deg_ref, p_ref, dinv_ref, g_ref):
  dcol = deg_ref[0, :, 0:1] + deg_ref[1, :, 0:1] + 1.0
  dinv = lax.rsqrt(dcol[:N])
  dinv_ref[...] = dinv
  g_ref[...] = p_ref[...] * dinv


def _tc_dinv_scale(deg, p1):
  return pl.pallas_call(
      _dinv_scale_body,
      out_shape=(
          jax.ShapeDtypeStruct((N, 1), jnp.float32),
          jax.ShapeDtypeStruct((N, HID), jnp.float32),
      ),
  )(deg, p1)


def _mid_body(s_ref, g_ref, dinv_ref, b_ref, w_ref, o_ref):
  dinv = dinv_ref[...]
  h = dinv * (s_ref[0, :N, :] + s_ref[1, :N, :] + g_ref[...]) + b_ref[...]
  a = jnp.maximum(h, 0.0) * dinv
  o_ref[...] = jnp.dot(a, w_ref[...], preferred_element_type=jnp.float32)


def _tc_mid(s, g, dinv, b, w):
  return pl.pallas_call(
      _mid_body,
      out_shape=jax.ShapeDtypeStruct((N, w.shape[1]), jnp.float32),
  )(s, g, dinv, b.reshape(1, -1), w)


def _final_body(s_ref, g_ref, dinv_ref, b_ref, o_ref):
  o_ref[...] = dinv_ref[...] * (
      s_ref[0, :N, :] + s_ref[1, :N, :] + g_ref[...]) + b_ref[...]


def _tc_final(s, g, dinv, b):
  return pl.pallas_call(
      _final_body,
      out_shape=jax.ShapeDtypeStruct((N, NUM_CLASSES), jnp.float32),
  )(s, g, dinv, b.reshape(1, -1))


@jax.jit
def kernel(x, edge_index, W1, b1, W2, b2, W3, b3):
  src = edge_index[0]
  dst = edge_index[1]
  pad = EPAD - E
  src_p = jnp.concatenate([src, jnp.zeros((pad,), jnp.int32)])
  dst_p = jnp.concatenate([dst, jnp.full((pad,), DUMMY_DST, jnp.int32)])
  src_r = src_p.reshape(32, NCHUNKS, CHUNK)
  dst_r = dst_p.reshape(32, NCHUNKS, CHUNK)

  deg = _make_degree_kernel()(dst_r)
  p1 = _tc_matmul(x, W1)
  dinv, g1 = _tc_dinv_scale(deg, p1)

  s1 = _make_scatter(128)(g1, src_r, dst_r)
  g2 = _tc_mid(s1, g1, dinv, b1, W2)

  s2 = _make_scatter(64)(g2, src_r, dst_r)
  g3 = _tc_mid(s2, g2, dinv, b2, W3)

  s3 = _make_scatter(16)(g3, src_r, dst_r)
  return _tc_final(s3, g3, dinv, b3)
